# baseline (speedup 1.0000x reference)
import functools

import jax
import jax.numpy as jnp
from jax import lax
from jax.experimental import pallas as pl
from jax.experimental.pallas import tpu as pltpu
from jax.experimental.pallas import tpu_sc as plsc

_MAX_SEQ_LEN = 8192
_EMBED = 1024
_BATCH = 4
_SEQ = 4096

_NC = 2
_NS = 16
_NW = _NC * _NS
_ROWS_PER_W = _SEQ // _NW
_CHUNK = 32
_NCHUNK = _ROWS_PER_W // _CHUNK


def _make_sc_kernel():
    mesh = plsc.VectorSubcoreMesh(core_axis_name="c", subcore_axis_name="s")

    @functools.partial(
        pl.kernel,
        mesh=mesh,
        out_type=jax.ShapeDtypeStruct((_BATCH, _SEQ, _EMBED), jnp.float32),
        scratch_types=[pltpu.SemaphoreType.DMA],
    )
    def pos_embed_broadcast(table_hbm, out_hbm, sem):
        wid = lax.axis_index("s") * _NC + lax.axis_index("c")
        base = wid * _ROWS_PER_W
        handles = [
            pltpu.async_copy(
                table_hbm.at[pl.ds(base, _ROWS_PER_W)],
                out_hbm.at[b, pl.ds(base, _ROWS_PER_W)],
                sem,
            )
            for b in range(_BATCH)
        ]
        for h in handles:
            h.wait()

    return pos_embed_broadcast


_sc_kernel = _make_sc_kernel()


def kernel(x, table):
    del x
    return _sc_kernel(table)


# speedup vs baseline: 44.9893x; 44.9893x over previous
import functools

import jax
import jax.numpy as jnp
from jax import lax
from jax.experimental import pallas as pl
from jax.experimental.pallas import tpu as pltpu
from jax.experimental.pallas import tpu_sc as plsc

_MAX_SEQ_LEN = 8192
_EMBED = 1024
_BATCH = 4
_SEQ = 4096

_NC = 2
_NS = 16
_NW = _NC * _NS
_ROWS_PER_W = _SEQ // _NW
_CHUNK = 64
_NCHUNK = _ROWS_PER_W // _CHUNK


def _make_sc_kernel():
    mesh = plsc.VectorSubcoreMesh(core_axis_name="c", subcore_axis_name="s")

    @functools.partial(
        pl.kernel,
        mesh=mesh,
        out_type=jax.ShapeDtypeStruct((_BATCH, _SEQ, _EMBED), jnp.float32),
        scratch_types=[pltpu.VMEM((_CHUNK, _EMBED), jnp.float32)],
    )
    def pos_embed_broadcast(table_hbm, out_hbm, buf):
        wid = lax.axis_index("s") * _NC + lax.axis_index("c")
        base = wid * _ROWS_PER_W
        for c in range(_NCHUNK):
            r0 = base + c * _CHUNK
            pltpu.sync_copy(table_hbm.at[pl.ds(r0, _CHUNK)], buf)
            for b in range(_BATCH):
                pltpu.sync_copy(buf, out_hbm.at[b, pl.ds(r0, _CHUNK)])

    return pos_embed_broadcast


_sc_kernel = _make_sc_kernel()


def kernel(x, table):
    del x
    return _sc_kernel(table)
